# baseline (speedup 1.0000x reference)
import functools

import jax
import jax.numpy as jnp
from jax import lax
from jax.experimental import pallas as pl
from jax.experimental.pallas import tpu as pltpu
from jax.experimental.pallas import tpu_sc as plsc

QS = 16384
NB_ROWS = 4096
ROW = 16 * 8 * 8
TAIL = QS - NB_ROWS
NW = 32
RPW = QS // NW
NBW = NB_ROWS // RPW


def _sc_store(bf, f, bl, lab, out, lab_out, sem, lsem):
    wid = lax.axis_index("s") * 2 + lax.axis_index("c")
    base = wid * RPW

    @pl.when(wid < NBW)
    def _():
        pltpu.async_copy(
            bf.at[pl.ds(base, RPW)], out.at[pl.ds(base, RPW)], sem
        ).wait()

    @pl.when(wid >= NBW)
    def _():
        pltpu.async_copy(
            f.at[pl.ds(base, RPW)], out.at[pl.ds(base, RPW)], sem
        ).wait()

    @pl.when(wid == 0)
    def _():
        pltpu.async_copy(bl, lab_out.at[pl.ds(0, NB_ROWS)], lsem).wait()

    @pl.when(wid == 1)
    def _():
        pltpu.async_copy(
            lab.at[pl.ds(NB_ROWS, TAIL)], lab_out.at[pl.ds(NB_ROWS, TAIL)], lsem
        ).wait()


def kernel(batch_features, batch_labels, features, labels):
    bf = batch_features.reshape(NB_ROWS, ROW)
    f = features.reshape(QS, ROW)
    mesh = plsc.VectorSubcoreMesh(core_axis_name="c", subcore_axis_name="s")
    run = functools.partial(
        pl.kernel,
        _sc_store,
        out_type=[
            jax.ShapeDtypeStruct((QS, ROW), jnp.float32),
            jax.ShapeDtypeStruct((QS,), jnp.int32),
        ],
        mesh=mesh,
        scratch_types=[pltpu.SemaphoreType.DMA, pltpu.SemaphoreType.DMA],
    )()
    out, lab_out = run(bf, f, batch_labels, labels)
    return out.reshape(QS, 16, 8, 8), lab_out


# speedup vs baseline: 11.0195x; 11.0195x over previous
import functools

import jax
import jax.numpy as jnp
from jax import lax
from jax.experimental import pallas as pl
from jax.experimental.pallas import tpu as pltpu
from jax.experimental.pallas import tpu_sc as plsc

QS = 16384
NB_ROWS = 4096
ROW = 16 * 8 * 8
TAIL = QS - NB_ROWS
NW = 32
RPW = QS // NW
NBW = NB_ROWS // RPW
CH = 32
NCH = RPW // CH


def _ring_copy(src, dst, base, b0, b1, in_sem, out_sem):
    bufs = (b0, b1)

    def in_copy(c, buf):
        return pltpu.make_async_copy(
            src.at[pl.ds(base + c * CH, CH)], bufs[buf], in_sem.at[buf]
        )

    def out_copy(c, buf):
        return pltpu.make_async_copy(
            bufs[buf], dst.at[pl.ds(base + c * CH, CH)], out_sem.at[buf]
        )

    in_copy(0, 0).start()
    for i in range(NCH):
        cur = i & 1
        nxt = 1 - cur
        if i + 1 < NCH:
            if i >= 1:
                out_copy(i - 1, nxt).wait()
            in_copy(i + 1, nxt).start()
        in_copy(i, cur).wait()
        out_copy(i, cur).start()
    out_copy(NCH - 2, (NCH - 2) & 1).wait()
    out_copy(NCH - 1, (NCH - 1) & 1).wait()


def _sc_store(bf, f, bl, lab, out, lab_out, b0, b1, in_sem, out_sem, lsem):
    wid = lax.axis_index("s") * 2 + lax.axis_index("c")
    base = wid * RPW

    @pl.when(wid < NBW)
    def _():
        _ring_copy(bf, out, base, b0, b1, in_sem, out_sem)

    @pl.when(wid >= NBW)
    def _():
        _ring_copy(f, out, base, b0, b1, in_sem, out_sem)

    @pl.when(wid == 0)
    def _():
        pltpu.async_copy(bl, lab_out.at[pl.ds(0, NB_ROWS)], lsem).wait()

    @pl.when(wid == 1)
    def _():
        pltpu.async_copy(
            lab.at[pl.ds(NB_ROWS, TAIL)], lab_out.at[pl.ds(NB_ROWS, TAIL)], lsem
        ).wait()


def kernel(batch_features, batch_labels, features, labels):
    bf = batch_features.reshape(NB_ROWS, ROW)
    f = features.reshape(QS, ROW)
    mesh = plsc.VectorSubcoreMesh(core_axis_name="c", subcore_axis_name="s")
    run = functools.partial(
        pl.kernel,
        _sc_store,
        out_type=[
            jax.ShapeDtypeStruct((QS, ROW), jnp.float32),
            jax.ShapeDtypeStruct((QS,), jnp.int32),
        ],
        mesh=mesh,
        scratch_types=[
            pltpu.VMEM((CH, ROW), jnp.float32),
            pltpu.VMEM((CH, ROW), jnp.float32),
            pltpu.SemaphoreType.DMA((2,)),
            pltpu.SemaphoreType.DMA((2,)),
            pltpu.SemaphoreType.DMA,
        ],
    )()
    out, lab_out = run(bf, f, batch_labels, labels)
    return out.reshape(QS, 16, 8, 8), lab_out


# speedup vs baseline: 11.1784x; 1.0144x over previous
import functools

import jax
import jax.numpy as jnp
from jax import lax
from jax.experimental import pallas as pl
from jax.experimental.pallas import tpu as pltpu
from jax.experimental.pallas import tpu_sc as plsc

QS = 16384
NB_ROWS = 4096
ROW = 16 * 8 * 8
TAIL = QS - NB_ROWS
NW = 32
RPW = QS // NW
NBW = NB_ROWS // RPW
CH = 32
NCH = RPW // CH


def _ring_copy(src, dst, base, b0, b1, in_sem, out_sem):
    bufs = (b0, b1)

    def in_copy(c, buf):
        return pltpu.make_async_copy(
            src.at[pl.ds(base + c * CH, CH)], bufs[buf], in_sem.at[buf]
        )

    def out_copy(c, buf):
        return pltpu.make_async_copy(
            bufs[buf], dst.at[pl.ds(base + c * CH, CH)], out_sem.at[buf]
        )

    in_copy(0, 0).start()
    for i in range(NCH):
        cur = i & 1
        nxt = 1 - cur
        if i + 1 < NCH:
            if i >= 1:
                out_copy(i - 1, nxt).wait()
            in_copy(i + 1, nxt).start()
        in_copy(i, cur).wait()
        out_copy(i, cur).start()
    out_copy(NCH - 2, (NCH - 2) & 1).wait()
    out_copy(NCH - 1, (NCH - 1) & 1).wait()


def _sc_store(bf, f, bl, lab, out, lab_out, s0, s1, in_sem, out_sem, lsem):
    sid = lax.axis_index("s")
    wid = sid * 2 + lax.axis_index("c")
    base = wid * RPW
    b0 = s0.at[sid]
    b1 = s1.at[sid]

    @pl.when(wid < NBW)
    def _():
        _ring_copy(bf, out, base, b0, b1, in_sem, out_sem)

    @pl.when(wid >= NBW)
    def _():
        _ring_copy(f, out, base, b0, b1, in_sem, out_sem)

    @pl.when(wid == 0)
    def _():
        pltpu.async_copy(bl, lab_out.at[pl.ds(0, NB_ROWS)], lsem).wait()

    @pl.when(wid == 1)
    def _():
        pltpu.async_copy(
            lab.at[pl.ds(NB_ROWS, TAIL)], lab_out.at[pl.ds(NB_ROWS, TAIL)], lsem
        ).wait()


def kernel(batch_features, batch_labels, features, labels):
    bf = batch_features.reshape(NB_ROWS, ROW)
    f = features.reshape(QS, ROW)
    mesh = plsc.VectorSubcoreMesh(core_axis_name="c", subcore_axis_name="s")
    run = functools.partial(
        pl.kernel,
        _sc_store,
        out_type=[
            jax.ShapeDtypeStruct((QS, ROW), jnp.float32),
            jax.ShapeDtypeStruct((QS,), jnp.int32),
        ],
        mesh=mesh,
        scratch_types=[
            pltpu.MemorySpace.VMEM_SHARED((16, CH, ROW), jnp.float32),
            pltpu.MemorySpace.VMEM_SHARED((16, CH, ROW), jnp.float32),
            pltpu.SemaphoreType.DMA((2,)),
            pltpu.SemaphoreType.DMA((2,)),
            pltpu.SemaphoreType.DMA,
        ],
    )()
    out, lab_out = run(bf, f, batch_labels, labels)
    return out.reshape(QS, 16, 8, 8), lab_out


# speedup vs baseline: 12.2506x; 1.0959x over previous
import jax
import jax.numpy as jnp
from jax.experimental import pallas as pl
from jax.experimental.pallas import tpu as pltpu

QS = 16384
NB_ROWS = 4096
ROW = 16 * 8 * 8
TAIL = QS - NB_ROWS
CH = 1024
NCH = QS // CH
NBC = NB_ROWS // CH
NBUF = 3


def _store_kernel(bf, f, bl, lab, out, lab_out, b0, b1, b2, in_sem, out_sem, lsem):
    bufs = (b0, b1, b2)

    def src(c):
        if c < NBC:
            return bf.at[pl.ds(c * CH, CH)]
        return f.at[pl.ds(c * CH, CH)]

    def in_copy(c):
        return pltpu.make_async_copy(src(c), bufs[c % NBUF], in_sem.at[c % NBUF])

    def out_copy(c):
        return pltpu.make_async_copy(
            bufs[c % NBUF], out.at[pl.ds(c * CH, CH)], out_sem.at[c % NBUF]
        )

    lab_in = pltpu.make_async_copy(bl, lab_out.at[pl.ds(0, NB_ROWS)], lsem.at[0])
    lab_tail = pltpu.make_async_copy(
        lab.at[pl.ds(NB_ROWS, TAIL)], lab_out.at[pl.ds(NB_ROWS, TAIL)], lsem.at[1]
    )
    lab_in.start()
    lab_tail.start()

    for c in range(NBUF - 1):
        in_copy(c).start()
    for c in range(NCH):
        if c + NBUF - 1 < NCH:
            if c >= 1:
                out_copy(c - 1).wait()
            in_copy(c + NBUF - 1).start()
        in_copy(c).wait()
        out_copy(c).start()
    for c in range(NCH - NBUF, NCH):
        out_copy(c).wait()
    lab_in.wait()
    lab_tail.wait()


def kernel(batch_features, batch_labels, features, labels):
    bf = batch_features.reshape(NB_ROWS, ROW)
    f = features.reshape(QS, ROW)
    out, lab_out = pl.pallas_call(
        _store_kernel,
        in_specs=[pl.BlockSpec(memory_space=pltpu.MemorySpace.HBM)] * 4,
        out_specs=[pl.BlockSpec(memory_space=pltpu.MemorySpace.HBM)] * 2,
        out_shape=[
            jax.ShapeDtypeStruct((QS, ROW), jnp.float32),
            jax.ShapeDtypeStruct((QS,), jnp.int32),
        ],
        scratch_shapes=[
            pltpu.VMEM((CH, ROW), jnp.float32),
            pltpu.VMEM((CH, ROW), jnp.float32),
            pltpu.VMEM((CH, ROW), jnp.float32),
            pltpu.SemaphoreType.DMA((NBUF,)),
            pltpu.SemaphoreType.DMA((NBUF,)),
            pltpu.SemaphoreType.DMA((2,)),
        ],
    )(bf, f, batch_labels, labels)
    return out.reshape(QS, 16, 8, 8), lab_out


# speedup vs baseline: 12.3518x; 1.0083x over previous
import jax
import jax.numpy as jnp
from jax.experimental import pallas as pl
from jax.experimental.pallas import tpu as pltpu

QS = 16384
NB_ROWS = 4096
ROW = 16 * 8 * 8
TAIL = QS - NB_ROWS
CH = 1024
NCH = QS // CH
NBC = NB_ROWS // CH
NBUF = 3


def _store_kernel(bf, f, bl, lab, lab_out, out_ref, b0, b1, b2, in_sem, lsem):
    i = pl.program_id(0)
    bufs = (b0, b1, b2)

    def start_in(c, buf, sem):
        @pl.when(c < NBC)
        def _():
            pltpu.make_async_copy(bf.at[pl.ds(c * CH, CH)], buf, sem).start()

        @pl.when(jnp.logical_and(c >= NBC, c < NCH))
        def _():
            pltpu.make_async_copy(f.at[pl.ds(c * CH, CH)], buf, sem).start()

    @pl.when(i == 0)
    def _():
        pltpu.make_async_copy(bl, lab_out.at[pl.ds(0, NB_ROWS)], lsem.at[0]).start()
        pltpu.make_async_copy(
            lab.at[pl.ds(NB_ROWS, TAIL)], lab_out.at[pl.ds(NB_ROWS, TAIL)], lsem.at[1]
        ).start()
        start_in(0, b0, in_sem.at[0])
        start_in(1, b1, in_sem.at[1])

    for k in range(NBUF):

        @pl.when((i + 2) % NBUF == k)
        def _(k=k):
            start_in(i + 2, bufs[k], in_sem.at[k])

    for k in range(NBUF):

        @pl.when(i % NBUF == k)
        def _(k=k):
            pltpu.make_async_copy(
                bf.at[pl.ds(0, CH)], bufs[k], in_sem.at[k]
            ).wait()
            out_ref[...] = bufs[k][...]

    @pl.when(i == NCH - 1)
    def _():
        pltpu.make_async_copy(bl, lab_out.at[pl.ds(0, NB_ROWS)], lsem.at[0]).wait()
        pltpu.make_async_copy(
            lab.at[pl.ds(NB_ROWS, TAIL)], lab_out.at[pl.ds(NB_ROWS, TAIL)], lsem.at[1]
        ).wait()


def kernel(batch_features, batch_labels, features, labels):
    bf = batch_features.reshape(NB_ROWS, ROW)
    f = features.reshape(QS, ROW)
    lab_out, out = pl.pallas_call(
        _store_kernel,
        grid=(NCH,),
        in_specs=[
            pl.BlockSpec(memory_space=pltpu.MemorySpace.HBM),
            pl.BlockSpec(memory_space=pltpu.MemorySpace.HBM),
            pl.BlockSpec(memory_space=pltpu.MemorySpace.HBM),
            pl.BlockSpec(memory_space=pltpu.MemorySpace.HBM),
        ],
        out_specs=[
            pl.BlockSpec(memory_space=pltpu.MemorySpace.HBM),
            pl.BlockSpec((CH, ROW), lambda i: (i, 0)),
        ],
        out_shape=[
            jax.ShapeDtypeStruct((QS,), jnp.int32),
            jax.ShapeDtypeStruct((QS, ROW), jnp.float32),
        ],
        scratch_shapes=[
            pltpu.VMEM((CH, ROW), jnp.float32),
            pltpu.VMEM((CH, ROW), jnp.float32),
            pltpu.VMEM((CH, ROW), jnp.float32),
            pltpu.SemaphoreType.DMA((NBUF,)),
            pltpu.SemaphoreType.DMA((2,)),
        ],
    )(bf, f, batch_labels, labels)
    return out.reshape(QS, 16, 8, 8), lab_out
